# baseline (speedup 1.0000x reference)
import jax
import jax.numpy as jnp
from jax.experimental import pallas as pl
import jax.experimental.pallas.tpu as pltpu

TOKENS = 8192
D_IN = 2048
D_OUT = 2048
BM = 1024
NC = 1024


def _matmul_kernel(x_ref, w_ref, o_ref):
    o_ref[...] = jax.lax.dot_general(
        x_ref[...],
        w_ref[...],
        dimension_numbers=(((1,), (1,)), ((), ())),
        precision=jax.lax.Precision.DEFAULT,
        preferred_element_type=jnp.float32,
    )


@jax.jit
def kernel(x, W):
    return pl.pallas_call(
        _matmul_kernel,
        grid=(D_OUT // NC, TOKENS // BM),
        in_specs=[
            pl.BlockSpec((BM, D_IN), lambda n, m: (m, 0)),
            pl.BlockSpec((NC, D_IN), lambda n, m: (n, 0)),
        ],
        out_specs=pl.BlockSpec((BM, NC), lambda n, m: (m, n)),
        out_shape=jax.ShapeDtypeStruct((TOKENS, D_OUT), jnp.float32),
        compiler_params=pltpu.CompilerParams(
            vmem_limit_bytes=62 * 1024 * 1024,
        ),
    )(x, W)


# speedup vs baseline: 1.0633x; 1.0633x over previous
import jax
import jax.numpy as jnp
from jax.experimental import pallas as pl
import jax.experimental.pallas.tpu as pltpu

TOKENS = 8192
D_IN = 2048
D_OUT = 2048
BM = 512


def _matmul_kernel(x_ref, w_ref, o_ref):
    o_ref[...] = jax.lax.dot_general(
        x_ref[...],
        w_ref[...],
        dimension_numbers=(((1,), (1,)), ((), ())),
        precision=jax.lax.Precision.DEFAULT,
        preferred_element_type=jnp.float32,
    )


@jax.jit
def kernel(x, W):
    return pl.pallas_call(
        _matmul_kernel,
        grid=(TOKENS // BM,),
        in_specs=[
            pl.BlockSpec((BM, D_IN), lambda i: (i, 0)),
            pl.BlockSpec((D_OUT, D_IN), lambda i: (0, 0)),
        ],
        out_specs=pl.BlockSpec((BM, D_OUT), lambda i: (i, 0)),
        out_shape=jax.ShapeDtypeStruct((TOKENS, D_OUT), jnp.float32),
        compiler_params=pltpu.CompilerParams(
            vmem_limit_bytes=62 * 1024 * 1024,
        ),
    )(x, W)


# speedup vs baseline: 1.0662x; 1.0028x over previous
import jax
import jax.numpy as jnp
from jax.experimental import pallas as pl
import jax.experimental.pallas.tpu as pltpu

TOKENS = 8192
D_IN = 2048
D_OUT = 2048
BM = 1024


def _matmul_kernel(x_ref, w_ref, o_ref):
    o_ref[...] = jax.lax.dot_general(
        x_ref[...],
        w_ref[...],
        dimension_numbers=(((1,), (1,)), ((), ())),
        precision=jax.lax.Precision.DEFAULT,
        preferred_element_type=jnp.float32,
    )


@jax.jit
def kernel(x, W):
    return pl.pallas_call(
        _matmul_kernel,
        grid=(TOKENS // BM,),
        in_specs=[
            pl.BlockSpec((BM, D_IN), lambda i: (i, 0)),
            pl.BlockSpec((D_OUT, D_IN), lambda i: (0, 0)),
        ],
        out_specs=pl.BlockSpec((BM, D_OUT), lambda i: (i, 0)),
        out_shape=jax.ShapeDtypeStruct((TOKENS, D_OUT), jnp.float32),
        compiler_params=pltpu.CompilerParams(
            vmem_limit_bytes=62 * 1024 * 1024,
        ),
    )(x, W)
